# Initial kernel scaffold; baseline (speedup 1.0000x reference)
import jax
import jax.numpy as jnp
from jax.experimental import pallas as pl


def kernel(edge_idx, edge_features, node_features, params):
    raise NotImplementedError("write your pallas kernel here")


# speedup vs baseline: 1.4501x; 1.4501x over previous
import functools

import jax
import jax.numpy as jnp
from jax import lax
from jax.experimental import pallas as pl
from jax.experimental.pallas import tpu as pltpu
from jax.experimental.pallas import tpu_sc as plsc

HID = 256
LN_EPS = 1e-5


def _mlp_tail(h, w2_ref, b2_ref, gm_ref, bt_ref):
    h = h * jax.nn.sigmoid(h)
    h = jnp.dot(h, w2_ref[...], preferred_element_type=jnp.float32) + b2_ref[...]
    if gm_ref is not None:
        mu = jnp.mean(h, axis=-1, keepdims=True)
        var = jnp.mean((h - mu) ** 2, axis=-1, keepdims=True)
        h = (h - mu) * lax.rsqrt(var + LN_EPS) * gm_ref[...] + bt_ref[...]
    return h


def _embed_body(x_ref, w1_ref, b1_ref, w2_ref, b2_ref, gm_ref, bt_ref, o_ref):
    h = jnp.dot(x_ref[...], w1_ref[...], preferred_element_type=jnp.float32)
    h = h + b1_ref[...]
    o_ref[...] = _mlp_tail(h, w2_ref, b2_ref, gm_ref, bt_ref)


def _pre_body(nf_ref, ws_ref, wr_ref, ns_ref, nr_ref):
    nf = nf_ref[...]
    ns_ref[...] = jnp.dot(nf, ws_ref[...], preferred_element_type=jnp.float32)
    nr_ref[...] = jnp.dot(nf, wr_ref[...], preferred_element_type=jnp.float32)


def _edge_body(ef_ref, g_ref, w1_ref, b1_ref, w2_ref, b2_ref, gm_ref, bt_ref,
               ne_ref, efo_ref):
    ef = ef_ref[...]
    h = jnp.dot(ef, w1_ref[...], preferred_element_type=jnp.float32)
    h = h + g_ref[...] + b1_ref[...]
    h = _mlp_tail(h, w2_ref, b2_ref, gm_ref, bt_ref)
    ne_ref[...] = h
    efo_ref[...] = ef + h


def _node_body(nf_ref, agg_ref, w1a_ref, w1b_ref, b1_ref, w2_ref, b2_ref,
               gm_ref, bt_ref, ws_ref, wr_ref, nfo_ref, ns_ref, nr_ref):
    nf = nf_ref[...]
    h = jnp.dot(nf, w1a_ref[...], preferred_element_type=jnp.float32)
    h = h + jnp.dot(agg_ref[...], w1b_ref[...], preferred_element_type=jnp.float32)
    h = h + b1_ref[...]
    h = _mlp_tail(h, w2_ref, b2_ref, gm_ref, bt_ref)
    nfo = nf + h
    nfo_ref[...] = nfo
    ns_ref[...] = jnp.dot(nfo, ws_ref[...], preferred_element_type=jnp.float32)
    nr_ref[...] = jnp.dot(nfo, wr_ref[...], preferred_element_type=jnp.float32)


def _out_body(nf_ref, w1_ref, b1_ref, w2_ref, b2_ref, o_ref):
    h = jnp.dot(nf_ref[...], w1_ref[...], preferred_element_type=jnp.float32)
    h = h + b1_ref[...]
    o_ref[...] = _mlp_tail(h, w2_ref, b2_ref, None, None)


def _row_spec(blk, d):
    return pl.BlockSpec((blk, d), lambda i: (i, 0))


def _full_spec(shape):
    nd = len(shape)
    return pl.BlockSpec(shape, lambda i: (0,) * nd)


def _pick_block(n, want):
    if n % want == 0:
        return want
    b = min(n, want)
    while n % b != 0:
        b -= 1
    return b


def _embed_mlp(x, p):
    e, d_in = x.shape
    blk = _pick_block(e, 2000)
    return pl.pallas_call(
        _embed_body,
        grid=(e // blk,),
        in_specs=[
            _row_spec(blk, d_in),
            _full_spec((d_in, HID)), _full_spec((1, HID)),
            _full_spec((HID, HID)), _full_spec((1, HID)),
            _full_spec((1, HID)), _full_spec((1, HID)),
        ],
        out_specs=_row_spec(blk, HID),
        out_shape=jax.ShapeDtypeStruct((e, HID), jnp.float32),
    )(x, p['W1'], p['b1'].reshape(1, -1), p['W2'], p['b2'].reshape(1, -1),
      p['g'].reshape(1, -1), p['bt'].reshape(1, -1))


def _pre_transform(nf, ws, wr):
    n = nf.shape[0]
    blk = _pick_block(n, 2000)
    return pl.pallas_call(
        _pre_body,
        grid=(n // blk,),
        in_specs=[_row_spec(blk, HID), _full_spec((HID, HID)),
                  _full_spec((HID, HID))],
        out_specs=[_row_spec(blk, HID), _row_spec(blk, HID)],
        out_shape=[jax.ShapeDtypeStruct((n, HID), jnp.float32),
                   jax.ShapeDtypeStruct((n, HID), jnp.float32)],
    )(nf, ws, wr)


def _edge_step(ef, g, w1e, p):
    e = ef.shape[0]
    blk = _pick_block(e, 2000)
    return pl.pallas_call(
        _edge_body,
        grid=(e // blk,),
        in_specs=[
            _row_spec(blk, HID), _row_spec(blk, HID),
            _full_spec((HID, HID)), _full_spec((1, HID)),
            _full_spec((HID, HID)), _full_spec((1, HID)),
            _full_spec((1, HID)), _full_spec((1, HID)),
        ],
        out_specs=[_row_spec(blk, HID), _row_spec(blk, HID)],
        out_shape=[jax.ShapeDtypeStruct((e, HID), jnp.float32),
                   jax.ShapeDtypeStruct((e, HID), jnp.float32)],
    )(ef, g, w1e, p['b1'].reshape(1, -1), p['W2'], p['b2'].reshape(1, -1),
      p['g'].reshape(1, -1), p['bt'].reshape(1, -1))


def _node_step(nf, agg, p, ws_next, wr_next):
    n = nf.shape[0]
    blk = _pick_block(n, 2000)
    w1a = p['W1'][:HID]
    w1b = p['W1'][HID:]
    return pl.pallas_call(
        _node_body,
        grid=(n // blk,),
        in_specs=[
            _row_spec(blk, HID), _row_spec(blk, HID),
            _full_spec((HID, HID)), _full_spec((HID, HID)),
            _full_spec((1, HID)),
            _full_spec((HID, HID)), _full_spec((1, HID)),
            _full_spec((1, HID)), _full_spec((1, HID)),
            _full_spec((HID, HID)), _full_spec((HID, HID)),
        ],
        out_specs=[_row_spec(blk, HID), _row_spec(blk, HID),
                   _row_spec(blk, HID)],
        out_shape=[jax.ShapeDtypeStruct((n, HID), jnp.float32),
                   jax.ShapeDtypeStruct((n, HID), jnp.float32),
                   jax.ShapeDtypeStruct((n, HID), jnp.float32)],
    )(nf, agg, w1a, w1b, p['b1'].reshape(1, -1), p['W2'],
      p['b2'].reshape(1, -1), p['g'].reshape(1, -1), p['bt'].reshape(1, -1),
      ws_next, wr_next)


def _out_mlp(nf, p):
    n = nf.shape[0]
    blk = _pick_block(n, 2000)
    return pl.pallas_call(
        _out_body,
        grid=(n // blk,),
        in_specs=[
            _row_spec(blk, HID),
            _full_spec((HID, HID)), _full_spec((1, HID)),
            _full_spec((HID, HID)), _full_spec((1, HID)),
        ],
        out_specs=_row_spec(blk, HID),
        out_shape=jax.ShapeDtypeStruct((n, HID), jnp.float32),
    )(nf, p['W1'], p['b1'].reshape(1, -1), p['W2'], p['b2'].reshape(1, -1))


_NC = 2
_NS = 16
_NW = _NC * _NS
_LANES = 16
_CHUNK = 80


def _vmem_add(dst_ref, src_ref, rows):
    def body(r, _):
        for k in range(HID // _LANES):
            sl = pl.ds(k * _LANES, _LANES)
            dst_ref[r, sl] = dst_ref[r, sl] + src_ref[r, sl]
        return 0
    lax.fori_loop(0, rows, body, 0, unroll=False)


def _gather_add(ns, nr, send, recv):
    e = send.shape[0]
    per_w = e // _NW
    n_full = per_w // _CHUNK
    tail = per_w - n_full * _CHUNK

    mesh = plsc.VectorSubcoreMesh(core_axis_name="c", subcore_axis_name="s")

    @functools.partial(
        pl.kernel, mesh=mesh,
        out_type=jax.ShapeDtypeStruct((e, HID), jnp.float32),
        scratch_types=[
            pltpu.VMEM((_CHUNK,), jnp.int32),
            pltpu.VMEM((_CHUNK,), jnp.int32),
            pltpu.VMEM((_CHUNK, HID), jnp.float32),
            pltpu.VMEM((_CHUNK, HID), jnp.float32),
            pltpu.VMEM((tail,), jnp.int32),
            pltpu.VMEM((tail,), jnp.int32),
            pltpu.VMEM((tail, HID), jnp.float32),
            pltpu.VMEM((tail, HID), jnp.float32),
            pltpu.SemaphoreType.DMA,
            pltpu.SemaphoreType.DMA,
        ],
    )
    def gather_kernel(ns_hbm, nr_hbm, send_hbm, recv_hbm, g_hbm,
                      ia, ib, ra, rb, ta, tb, tra, trb, sem, sem2):
        wid = lax.axis_index("s") * _NC + lax.axis_index("c")
        base = wid * per_w

        def chunk(j, _):
            off = base + j * _CHUNK
            pltpu.sync_copy(send_hbm.at[pl.ds(off, _CHUNK)], ia)
            pltpu.sync_copy(recv_hbm.at[pl.ds(off, _CHUNK)], ib)
            d1 = pltpu.async_copy(ns_hbm.at[ia], ra, sem)
            d2 = pltpu.async_copy(nr_hbm.at[ib], rb, sem2)
            d1.wait()
            d2.wait()
            _vmem_add(ra, rb, _CHUNK)
            pltpu.sync_copy(ra, g_hbm.at[pl.ds(off, _CHUNK)])
            return 0

        lax.fori_loop(0, n_full, chunk, 0, unroll=False)
        off = base + n_full * _CHUNK
        pltpu.sync_copy(send_hbm.at[pl.ds(off, tail)], ta)
        pltpu.sync_copy(recv_hbm.at[pl.ds(off, tail)], tb)
        d1 = pltpu.async_copy(ns_hbm.at[ta], tra, sem)
        d2 = pltpu.async_copy(nr_hbm.at[tb], trb, sem2)
        d1.wait()
        d2.wait()
        _vmem_add(tra, trb, tail)
        pltpu.sync_copy(tra, g_hbm.at[pl.ds(off, tail)])

    return gather_kernel(ns, nr, send, recv)


_N_PAD = 10240


def _seg_cumsum(ne, order):
    e = ne.shape[0]
    per_w = e // _NW
    n_full = per_w // _CHUNK
    tail = per_w - n_full * _CHUNK
    ngr = HID // _LANES

    mesh = plsc.VectorSubcoreMesh(core_axis_name="c", subcore_axis_name="s")

    @functools.partial(
        pl.kernel, mesh=mesh,
        out_type=[jax.ShapeDtypeStruct((e + 8, HID), jnp.float32),
                  jax.ShapeDtypeStruct((_NW, HID), jnp.float32)],
        scratch_types=[
            pltpu.VMEM((_CHUNK,), jnp.int32),
            pltpu.VMEM((_CHUNK, HID), jnp.float32),
            pltpu.VMEM((_CHUNK, HID), jnp.float32),
            pltpu.VMEM((tail,), jnp.int32),
            pltpu.VMEM((tail, HID), jnp.float32),
            pltpu.VMEM((tail, HID), jnp.float32),
            pltpu.VMEM((1, HID), jnp.float32),
            pltpu.SemaphoreType.DMA,
        ],
    )
    def cumsum_kernel(ne_hbm, ord_hbm, cs_hbm, tot_hbm,
                      ib, rin, rout, tb, trin, trout, carry_v, sem):
        wid = lax.axis_index("s") * _NC + lax.axis_index("c")
        base = wid * per_w

        zero = jnp.zeros((_LANES,), jnp.float32)
        for k in range(ngr):
            carry_v[0, pl.ds(k * _LANES, _LANES)] = zero

        def run(in_ref, out_ref, nrows):
            def row(r, _):
                for k in range(ngr):
                    sl = pl.ds(k * _LANES, _LANES)
                    cv = carry_v[0, sl]
                    out_ref[r, sl] = cv
                    carry_v[0, sl] = cv + in_ref[r, sl]
                return 0
            lax.fori_loop(0, nrows, row, 0, unroll=False)

        def chunk(j, _):
            off = base + j * _CHUNK
            pltpu.sync_copy(ord_hbm.at[pl.ds(off, _CHUNK)], ib)
            pltpu.async_copy(ne_hbm.at[ib], rin, sem).wait()
            run(rin, rout, _CHUNK)
            pltpu.sync_copy(rout, cs_hbm.at[pl.ds(off, _CHUNK)])
            return 0

        lax.fori_loop(0, n_full, chunk, 0, unroll=False)
        off = base + n_full * _CHUNK
        pltpu.sync_copy(ord_hbm.at[pl.ds(off, tail)], tb)
        pltpu.async_copy(ne_hbm.at[tb], trin, sem).wait()
        run(trin, trout, tail)
        pltpu.sync_copy(trout, cs_hbm.at[pl.ds(off, tail)])
        pltpu.sync_copy(carry_v, tot_hbm.at[pl.ds(wid, 1)])
        @pl.when(wid == 0)
        def _():
            def zrow(r, _):
                for k in range(ngr):
                    trout[r, pl.ds(k * _LANES, _LANES)] = zero
                return 0
            lax.fori_loop(0, 8, zrow, 0, unroll=False)
            pltpu.sync_copy(trout.at[pl.ds(0, 8)], cs_hbm.at[pl.ds(e, 8)])

    return cumsum_kernel(ne, order)


def _seg_offsets(tot):
    ngr = HID // _LANES
    mesh = plsc.VectorSubcoreMesh(core_axis_name="c", subcore_axis_name="s")

    @functools.partial(
        pl.kernel, mesh=mesh,
        out_type=jax.ShapeDtypeStruct((40, HID), jnp.float32),
        scratch_types=[
            pltpu.VMEM((_NW, HID), jnp.float32),
            pltpu.VMEM((40, HID), jnp.float32),
        ],
    )
    def offsets_kernel(tot_hbm, off_hbm, tot_v, off_v):
        wid = lax.axis_index("s") * _NC + lax.axis_index("c")

        @pl.when(wid == 0)
        def _():
            pltpu.sync_copy(tot_hbm, tot_v)
            for k in range(ngr):
                sl = pl.ds(k * _LANES, _LANES)
                acc = jnp.zeros((_LANES,), jnp.float32)
                for t in range(_NW):
                    off_v[t, sl] = acc
                    acc = acc + tot_v[t, sl]
                off_v[_NW, sl] = acc
                for t in range(_NW + 1, 40):
                    off_v[t, sl] = acc
            pltpu.sync_copy(off_v, off_hbm)

    return offsets_kernel(tot)


def _seg_boundary(cs, off, rp_start, rp_end, per_w):
    nodes_per_t = _N_PAD // _NW
    n_sub = nodes_per_t // _CHUNK
    ngr = HID // _LANES

    mesh = plsc.VectorSubcoreMesh(core_axis_name="c", subcore_axis_name="s")

    @functools.partial(
        pl.kernel, mesh=mesh,
        out_type=jax.ShapeDtypeStruct((_N_PAD, HID), jnp.float32),
        scratch_types=[
            pltpu.VMEM((_CHUNK,), jnp.int32),
            pltpu.VMEM((_CHUNK,), jnp.int32),
            pltpu.VMEM((_CHUNK,), jnp.int32),
            pltpu.VMEM((_CHUNK,), jnp.int32),
            pltpu.VMEM((_CHUNK, HID), jnp.float32),
            pltpu.VMEM((_CHUNK, HID), jnp.float32),
            pltpu.VMEM((_CHUNK, HID), jnp.float32),
            pltpu.VMEM((_CHUNK, HID), jnp.float32),
            pltpu.SemaphoreType.DMA,
        ],
    )
    def boundary_kernel(cs_hbm, off_hbm, rs_hbm, re_hbm, agg_hbm,
                        ia, ie, ta, te, ga, gb, oa, ob, sem):
        wid = lax.axis_index("s") * _NC + lax.axis_index("c")
        pw = jnp.full((_LANES,), per_w, jnp.int32)

        def sub(q, _):
            nbase = wid * nodes_per_t + q * _CHUNK
            pltpu.sync_copy(rs_hbm.at[pl.ds(nbase, _CHUNK)], ia)
            pltpu.sync_copy(re_hbm.at[pl.ds(nbase, _CHUNK)], ie)
            for g in range(_CHUNK // _LANES):
                sl = pl.ds(g * _LANES, _LANES)
                ta[sl] = lax.div(ia[sl], pw)
                te[sl] = lax.div(ie[sl], pw)
            pltpu.async_copy(cs_hbm.at[ia], ga, sem).wait()
            pltpu.async_copy(cs_hbm.at[ie], gb, sem).wait()
            pltpu.async_copy(off_hbm.at[ta], oa, sem).wait()
            pltpu.async_copy(off_hbm.at[te], ob, sem).wait()

            def row(r, _):
                for k in range(ngr):
                    slk = pl.ds(k * _LANES, _LANES)
                    ga[r, slk] = (gb[r, slk] + ob[r, slk]) - (
                        ga[r, slk] + oa[r, slk])
                return 0
            lax.fori_loop(0, _CHUNK, row, 0, unroll=False)
            pltpu.sync_copy(ga, agg_hbm.at[pl.ds(nbase, _CHUNK)])
            return 0

        lax.fori_loop(0, n_sub, sub, 0, unroll=False)

    return boundary_kernel(cs, off, rp_start, rp_end)


def _segment_sum(ne, order, rp_start, rp_end):
    e = ne.shape[0]
    cs, tot = _seg_cumsum(ne, order)
    off = _seg_offsets(tot)
    return _seg_boundary(cs, off, rp_start, rp_end, e // _NW)


def kernel(edge_idx, edge_features, node_features, params):
    send = edge_idx[0]
    recv = edge_idx[1]
    n = node_features.shape[0]

    order = jnp.argsort(recv).astype(jnp.int32)
    recv_sorted = jnp.take(recv, order)
    rowptr = jnp.searchsorted(
        recv_sorted, jnp.arange(_N_PAD + 1, dtype=jnp.int32)).astype(jnp.int32)
    rp_start = rowptr[:_N_PAD]
    rp_end = rowptr[1:]

    ef = _embed_mlp(edge_features, params['embed_edge'])
    nf = node_features
    w1 = params['step0']['edge']['W1']
    ns, nr = _pre_transform(nf, w1[HID:2 * HID], w1[2 * HID:])
    for i in range(4):
        p = params['step%d' % i]
        w1 = p['edge']['W1']
        g = _gather_add(ns, nr, send, recv)
        ne, ef = _edge_step(ef, g, w1[:HID], p['edge'])
        agg = _segment_sum(ne, order, rp_start, rp_end)[:n]
        w1n = params['step%d' % min(i + 1, 3)]['edge']['W1']
        nf, ns, nr = _node_step(nf, agg, p['node'],
                                w1n[HID:2 * HID], w1n[2 * HID:])
    nf = _out_mlp(nf, params['node_out'])
    return (ef, nf)


# speedup vs baseline: 1.8017x; 1.2425x over previous
import functools

import jax
import jax.numpy as jnp
from jax import lax
from jax.experimental import pallas as pl
from jax.experimental.pallas import tpu as pltpu
from jax.experimental.pallas import tpu_sc as plsc

HID = 256
LN_EPS = 1e-5


def _mlp_tail(h, w2_ref, b2_ref, gm_ref, bt_ref):
    h = h * jax.nn.sigmoid(h)
    h = jnp.dot(h, w2_ref[...], preferred_element_type=jnp.float32) + b2_ref[...]
    if gm_ref is not None:
        mu = jnp.mean(h, axis=-1, keepdims=True)
        var = jnp.mean((h - mu) ** 2, axis=-1, keepdims=True)
        h = (h - mu) * lax.rsqrt(var + LN_EPS) * gm_ref[...] + bt_ref[...]
    return h


def _embed_body(x_ref, w1_ref, b1_ref, w2_ref, b2_ref, gm_ref, bt_ref, o_ref):
    h = jnp.dot(x_ref[...], w1_ref[...], preferred_element_type=jnp.float32)
    h = h + b1_ref[...]
    o_ref[...] = _mlp_tail(h, w2_ref, b2_ref, gm_ref, bt_ref)


def _pre_body(nf_ref, ws_ref, wr_ref, ns_ref, nr_ref):
    nf = nf_ref[...]
    ns_ref[...] = jnp.dot(nf, ws_ref[...], preferred_element_type=jnp.float32)
    nr_ref[...] = jnp.dot(nf, wr_ref[...], preferred_element_type=jnp.float32)


def _edge_body(ef_ref, g_ref, w1_ref, b1_ref, w2_ref, b2_ref, gm_ref, bt_ref,
               ne_ref, efo_ref):
    ef = ef_ref[...]
    h = jnp.dot(ef, w1_ref[...], preferred_element_type=jnp.float32)
    h = h + g_ref[...] + b1_ref[...]
    h = _mlp_tail(h, w2_ref, b2_ref, gm_ref, bt_ref)
    ne_ref[...] = h
    efo_ref[...] = ef + h


def _node_body(nf_ref, agg_ref, w1a_ref, w1b_ref, b1_ref, w2_ref, b2_ref,
               gm_ref, bt_ref, ws_ref, wr_ref, nfo_ref, ns_ref, nr_ref):
    nf = nf_ref[...]
    h = jnp.dot(nf, w1a_ref[...], preferred_element_type=jnp.float32)
    h = h + jnp.dot(agg_ref[...], w1b_ref[...], preferred_element_type=jnp.float32)
    h = h + b1_ref[...]
    h = _mlp_tail(h, w2_ref, b2_ref, gm_ref, bt_ref)
    nfo = nf + h
    nfo_ref[...] = nfo
    ns_ref[...] = jnp.dot(nfo, ws_ref[...], preferred_element_type=jnp.float32)
    nr_ref[...] = jnp.dot(nfo, wr_ref[...], preferred_element_type=jnp.float32)


def _out_body(nf_ref, w1_ref, b1_ref, w2_ref, b2_ref, o_ref):
    h = jnp.dot(nf_ref[...], w1_ref[...], preferred_element_type=jnp.float32)
    h = h + b1_ref[...]
    o_ref[...] = _mlp_tail(h, w2_ref, b2_ref, None, None)


def _row_spec(blk, d):
    return pl.BlockSpec((blk, d), lambda i: (i, 0))


def _full_spec(shape):
    nd = len(shape)
    return pl.BlockSpec(shape, lambda i: (0,) * nd)


def _pick_block(n, want):
    if n % want == 0:
        return want
    b = min(n, want)
    while n % b != 0:
        b -= 1
    return b


def _embed_mlp(x, p):
    e, d_in = x.shape
    blk = _pick_block(e, 2000)
    return pl.pallas_call(
        _embed_body,
        grid=(e // blk,),
        in_specs=[
            _row_spec(blk, d_in),
            _full_spec((d_in, HID)), _full_spec((1, HID)),
            _full_spec((HID, HID)), _full_spec((1, HID)),
            _full_spec((1, HID)), _full_spec((1, HID)),
        ],
        out_specs=_row_spec(blk, HID),
        out_shape=jax.ShapeDtypeStruct((e, HID), jnp.float32),
    )(x, p['W1'], p['b1'].reshape(1, -1), p['W2'], p['b2'].reshape(1, -1),
      p['g'].reshape(1, -1), p['bt'].reshape(1, -1))


def _pre_transform(nf, ws, wr):
    n = nf.shape[0]
    blk = _pick_block(n, 2000)
    return pl.pallas_call(
        _pre_body,
        grid=(n // blk,),
        in_specs=[_row_spec(blk, HID), _full_spec((HID, HID)),
                  _full_spec((HID, HID))],
        out_specs=[_row_spec(blk, HID), _row_spec(blk, HID)],
        out_shape=[jax.ShapeDtypeStruct((n, HID), jnp.float32),
                   jax.ShapeDtypeStruct((n, HID), jnp.float32)],
    )(nf, ws, wr)


def _edge_step(ef, g, w1e, p):
    e = ef.shape[0]
    blk = _pick_block(e, 2000)
    return pl.pallas_call(
        _edge_body,
        grid=(e // blk,),
        in_specs=[
            _row_spec(blk, HID), _row_spec(blk, HID),
            _full_spec((HID, HID)), _full_spec((1, HID)),
            _full_spec((HID, HID)), _full_spec((1, HID)),
            _full_spec((1, HID)), _full_spec((1, HID)),
        ],
        out_specs=[_row_spec(blk, HID), _row_spec(blk, HID)],
        out_shape=[jax.ShapeDtypeStruct((e, HID), jnp.float32),
                   jax.ShapeDtypeStruct((e, HID), jnp.float32)],
    )(ef, g, w1e, p['b1'].reshape(1, -1), p['W2'], p['b2'].reshape(1, -1),
      p['g'].reshape(1, -1), p['bt'].reshape(1, -1))


def _node_step(nf, agg, p, ws_next, wr_next):
    n = nf.shape[0]
    blk = _pick_block(n, 2000)
    w1a = p['W1'][:HID]
    w1b = p['W1'][HID:]
    return pl.pallas_call(
        _node_body,
        grid=(n // blk,),
        in_specs=[
            _row_spec(blk, HID), _row_spec(blk, HID),
            _full_spec((HID, HID)), _full_spec((HID, HID)),
            _full_spec((1, HID)),
            _full_spec((HID, HID)), _full_spec((1, HID)),
            _full_spec((1, HID)), _full_spec((1, HID)),
            _full_spec((HID, HID)), _full_spec((HID, HID)),
        ],
        out_specs=[_row_spec(blk, HID), _row_spec(blk, HID),
                   _row_spec(blk, HID)],
        out_shape=[jax.ShapeDtypeStruct((n, HID), jnp.float32),
                   jax.ShapeDtypeStruct((n, HID), jnp.float32),
                   jax.ShapeDtypeStruct((n, HID), jnp.float32)],
    )(nf, agg, w1a, w1b, p['b1'].reshape(1, -1), p['W2'],
      p['b2'].reshape(1, -1), p['g'].reshape(1, -1), p['bt'].reshape(1, -1),
      ws_next, wr_next)


def _out_mlp(nf, p):
    n = nf.shape[0]
    blk = _pick_block(n, 2000)
    return pl.pallas_call(
        _out_body,
        grid=(n // blk,),
        in_specs=[
            _row_spec(blk, HID),
            _full_spec((HID, HID)), _full_spec((1, HID)),
            _full_spec((HID, HID)), _full_spec((1, HID)),
        ],
        out_specs=_row_spec(blk, HID),
        out_shape=jax.ShapeDtypeStruct((n, HID), jnp.float32),
    )(nf, p['W1'], p['b1'].reshape(1, -1), p['W2'], p['b2'].reshape(1, -1))


_NC = 2
_NS = 16
_NW = _NC * _NS
_LANES = 16
_CHUNK = 80


def _vmem_add(dst_ref, src_ref, rows):
    def body(r, _):
        for k in range(HID // _LANES):
            sl = pl.ds(k * _LANES, _LANES)
            dst_ref[r, sl] = dst_ref[r, sl] + src_ref[r, sl]
        return 0
    lax.fori_loop(0, rows, body, 0, unroll=False)


def _gather_add(ns, nr, send3, recv3, e, per_w, n_ch, tail):
    mesh = plsc.VectorSubcoreMesh(core_axis_name="c", subcore_axis_name="s")

    @functools.partial(
        pl.kernel, mesh=mesh,
        out_type=jax.ShapeDtypeStruct((e, HID), jnp.float32),
        scratch_types=[
            pltpu.VMEM((n_ch, _CHUNK), jnp.int32),
            pltpu.VMEM((n_ch, _CHUNK), jnp.int32),
            pltpu.VMEM((_CHUNK, HID), jnp.float32),
            pltpu.VMEM((_CHUNK, HID), jnp.float32),
            pltpu.VMEM((_CHUNK, HID), jnp.float32),
            pltpu.VMEM((_CHUNK, HID), jnp.float32),
            pltpu.SemaphoreType.DMA,
            pltpu.SemaphoreType.DMA,
            pltpu.SemaphoreType.DMA,
            pltpu.SemaphoreType.DMA,
        ],
    )
    def gather_kernel(ns_hbm, nr_hbm, send_hbm, recv_hbm, g_hbm,
                      iv_s, iv_r, ra0, rb0, ra1, rb1, s0, s1, s2, s3):
        wid = lax.axis_index("s") * _NC + lax.axis_index("c")
        base = wid * per_w
        bufs = ((ra0, rb0, s0, s1), (ra1, rb1, s2, s3))

        pltpu.sync_copy(send_hbm.at[wid], iv_s)
        pltpu.sync_copy(recv_hbm.at[wid], iv_r)

        def issue(j, b):
            ra, rb, sa, sb = bufs[b]
            pltpu.async_copy(ns_hbm.at[iv_s.at[j]], ra, sa)
            pltpu.async_copy(nr_hbm.at[iv_r.at[j]], rb, sb)

        def consume(j, b, rows):
            ra, rb, sa, sb = bufs[b]
            pltpu.make_async_copy(ns_hbm.at[iv_s.at[j]], ra, sa).wait()
            pltpu.make_async_copy(nr_hbm.at[iv_r.at[j]], rb, sb).wait()
            _vmem_add(ra, rb, rows)
            if rows == _CHUNK:
                pltpu.sync_copy(ra, g_hbm.at[pl.ds(base + j * _CHUNK, _CHUNK)])
            else:
                pltpu.sync_copy(ra.at[pl.ds(0, rows)],
                                g_hbm.at[pl.ds(base + j * _CHUNK, rows)])

        issue(0, 0)
        issue(1, 1)

        def pair(jj, _):
            j0 = 2 * jj
            consume(j0, 0, _CHUNK)
            @pl.when(j0 + 2 < n_ch)
            def _():
                issue(j0 + 2, 0)
            consume(j0 + 1, 1, _CHUNK)
            @pl.when(j0 + 3 < n_ch)
            def _():
                issue(j0 + 3, 1)
            return 0

        lax.fori_loop(0, (n_ch - 1) // 2, pair, 0, unroll=False)
        consume(n_ch - 1, (n_ch - 1) % 2, tail)

    return gather_kernel(ns, nr, send3, recv3)


_N_PAD = 10240


def _seg_cumsum(ne, order3, e, per_w, n_ch, tail):
    ngr = HID // _LANES
    mesh = plsc.VectorSubcoreMesh(core_axis_name="c", subcore_axis_name="s")

    @functools.partial(
        pl.kernel, mesh=mesh,
        out_type=[jax.ShapeDtypeStruct((e + 8, HID), jnp.float32),
                  jax.ShapeDtypeStruct((_NW, HID), jnp.float32)],
        scratch_types=[
            pltpu.VMEM((n_ch, _CHUNK), jnp.int32),
            pltpu.VMEM((_CHUNK, HID), jnp.float32),
            pltpu.VMEM((_CHUNK, HID), jnp.float32),
            pltpu.VMEM((_CHUNK, HID), jnp.float32),
            pltpu.VMEM((1, HID), jnp.float32),
            pltpu.SemaphoreType.DMA,
            pltpu.SemaphoreType.DMA,
        ],
    )
    def cumsum_kernel(ne_hbm, ord_hbm, cs_hbm, tot_hbm,
                      iv, rin0, rin1, rout, carry_v, s0, s1):
        wid = lax.axis_index("s") * _NC + lax.axis_index("c")
        base = wid * per_w
        bufs = ((rin0, s0), (rin1, s1))

        pltpu.sync_copy(ord_hbm.at[wid], iv)

        zero = jnp.zeros((_LANES,), jnp.float32)
        for k in range(ngr):
            carry_v[0, pl.ds(k * _LANES, _LANES)] = zero

        def issue(j, b):
            rin, sa = bufs[b]
            pltpu.async_copy(ne_hbm.at[iv.at[j]], rin, sa)

        def consume(j, b, rows):
            rin, sa = bufs[b]
            pltpu.make_async_copy(ne_hbm.at[iv.at[j]], rin, sa).wait()
            for k in range(ngr):
                sl = pl.ds(k * _LANES, _LANES)

                def rowbody(r, cv):
                    rout[r, sl] = cv
                    return cv + rin[r, sl]

                cv_end = lax.fori_loop(0, rows, rowbody, carry_v[0, sl],
                                       unroll=8)
                carry_v[0, sl] = cv_end
            if rows == _CHUNK:
                pltpu.sync_copy(rout, cs_hbm.at[pl.ds(base + j * _CHUNK,
                                                      _CHUNK)])
            else:
                pltpu.sync_copy(rout.at[pl.ds(0, rows)],
                                cs_hbm.at[pl.ds(base + j * _CHUNK, rows)])

        issue(0, 0)
        issue(1, 1)

        def pair(jj, _):
            j0 = 2 * jj
            consume(j0, 0, _CHUNK)
            @pl.when(j0 + 2 < n_ch)
            def _():
                issue(j0 + 2, 0)
            consume(j0 + 1, 1, _CHUNK)
            @pl.when(j0 + 3 < n_ch)
            def _():
                issue(j0 + 3, 1)
            return 0

        lax.fori_loop(0, (n_ch - 1) // 2, pair, 0, unroll=False)
        consume(n_ch - 1, (n_ch - 1) % 2, tail)

        pltpu.sync_copy(carry_v, tot_hbm.at[pl.ds(wid, 1)])
        @pl.when(wid == 0)
        def _():
            def zrow(r, _):
                for k in range(ngr):
                    rout[r, pl.ds(k * _LANES, _LANES)] = zero
                return 0
            lax.fori_loop(0, 8, zrow, 0, unroll=False)
            pltpu.sync_copy(rout.at[pl.ds(0, 8)], cs_hbm.at[pl.ds(e, 8)])

    return cumsum_kernel(ne, order3)


def _offsets_body(tot_ref, off_ref):
    acc = jnp.zeros((1, HID), jnp.float32)
    for t in range(_NW):
        off_ref[pl.ds(t, 1), :] = acc
        acc = acc + tot_ref[pl.ds(t, 1), :]
    for t in range(_NW, 40):
        off_ref[pl.ds(t, 1), :] = acc


def _seg_offsets(tot):
    return pl.pallas_call(
        _offsets_body,
        grid=(1,),
        in_specs=[_full_spec((_NW, HID))],
        out_specs=_full_spec((40, HID)),
        out_shape=jax.ShapeDtypeStruct((40, HID), jnp.float32),
    )(tot)


def _seg_boundary(cs, off, rp_start, rp_end, per_w):
    nodes_per_t = _N_PAD // _NW
    n_sub = nodes_per_t // _CHUNK
    ngr = HID // _LANES

    mesh = plsc.VectorSubcoreMesh(core_axis_name="c", subcore_axis_name="s")

    @functools.partial(
        pl.kernel, mesh=mesh,
        out_type=jax.ShapeDtypeStruct((_N_PAD, HID), jnp.float32),
        scratch_types=[
            pltpu.VMEM((_CHUNK,), jnp.int32),
            pltpu.VMEM((_CHUNK,), jnp.int32),
            pltpu.VMEM((_CHUNK,), jnp.int32),
            pltpu.VMEM((_CHUNK,), jnp.int32),
            pltpu.VMEM((_CHUNK, HID), jnp.float32),
            pltpu.VMEM((_CHUNK, HID), jnp.float32),
            pltpu.VMEM((_CHUNK, HID), jnp.float32),
            pltpu.VMEM((_CHUNK, HID), jnp.float32),
            pltpu.SemaphoreType.DMA,
            pltpu.SemaphoreType.DMA,
            pltpu.SemaphoreType.DMA,
            pltpu.SemaphoreType.DMA,
        ],
    )
    def boundary_kernel(cs_hbm, off_hbm, rs_hbm, re_hbm, agg_hbm,
                        ia, ie, ta, te, ga, gb, oa, ob, s0, s1, s2, s3):
        wid = lax.axis_index("s") * _NC + lax.axis_index("c")
        pw = jnp.full((_LANES,), per_w, jnp.int32)

        def sub(q, _):
            nbase = wid * nodes_per_t + q * _CHUNK
            pltpu.sync_copy(rs_hbm.at[pl.ds(nbase, _CHUNK)], ia)
            pltpu.sync_copy(re_hbm.at[pl.ds(nbase, _CHUNK)], ie)
            for g in range(_CHUNK // _LANES):
                sl = pl.ds(g * _LANES, _LANES)
                ta[sl] = lax.div(ia[sl], pw)
                te[sl] = lax.div(ie[sl], pw)
            d0 = pltpu.async_copy(cs_hbm.at[ia], ga, s0)
            d1 = pltpu.async_copy(cs_hbm.at[ie], gb, s1)
            d2 = pltpu.async_copy(off_hbm.at[ta], oa, s2)
            d3 = pltpu.async_copy(off_hbm.at[te], ob, s3)
            d0.wait()
            d1.wait()
            d2.wait()
            d3.wait()

            def row(r, _):
                for k in range(ngr):
                    slk = pl.ds(k * _LANES, _LANES)
                    ga[r, slk] = (gb[r, slk] + ob[r, slk]) - (
                        ga[r, slk] + oa[r, slk])
                return 0
            lax.fori_loop(0, _CHUNK, row, 0, unroll=False)
            pltpu.sync_copy(ga, agg_hbm.at[pl.ds(nbase, _CHUNK)])
            return 0

        lax.fori_loop(0, n_sub, sub, 0, unroll=False)

    return boundary_kernel(cs, off, rp_start, rp_end)


def _segment_sum(ne, order3, rp_start, rp_end, e, per_w, n_ch, tail):
    cs, tot = _seg_cumsum(ne, order3, e, per_w, n_ch, tail)
    off = _seg_offsets(tot)
    return _seg_boundary(cs, off, rp_start, rp_end, per_w)


def kernel(edge_idx, edge_features, node_features, params):
    send = edge_idx[0]
    recv = edge_idx[1]
    n = node_features.shape[0]

    e = send.shape[0]
    per_w = e // _NW
    n_ch = -(-per_w // _CHUNK)
    tail = per_w - (n_ch - 1) * _CHUNK
    pad = n_ch * _CHUNK - per_w

    def _blocks(idx):
        return jnp.pad(idx.reshape(_NW, per_w),
                       ((0, 0), (0, pad))).reshape(_NW, n_ch, _CHUNK)

    order = jnp.argsort(recv).astype(jnp.int32)
    recv_sorted = jnp.take(recv, order)
    rowptr = jnp.searchsorted(
        recv_sorted, jnp.arange(_N_PAD + 1, dtype=jnp.int32)).astype(jnp.int32)
    rp_start = rowptr[:_N_PAD]
    rp_end = rowptr[1:]
    send3 = _blocks(send)
    recv3 = _blocks(recv)
    order3 = _blocks(order)

    ef = _embed_mlp(edge_features, params['embed_edge'])
    nf = node_features
    w1 = params['step0']['edge']['W1']
    ns, nr = _pre_transform(nf, w1[HID:2 * HID], w1[2 * HID:])
    for i in range(4):
        p = params['step%d' % i]
        w1 = p['edge']['W1']
        g = _gather_add(ns, nr, send3, recv3, e, per_w, n_ch, tail)
        ne, ef = _edge_step(ef, g, w1[:HID], p['edge'])
        agg = _segment_sum(ne, order3, rp_start, rp_end,
                           e, per_w, n_ch, tail)[:n]
        w1n = params['step%d' % min(i + 1, 3)]['edge']['W1']
        nf, ns, nr = _node_step(nf, agg, p['node'],
                                w1n[HID:2 * HID], w1n[2 * HID:])
    nf = _out_mlp(nf, params['node_out'])
    return (ef, nf)


# speedup vs baseline: 2.2152x; 1.2295x over previous
import functools

import jax
import jax.numpy as jnp
from jax import lax
from jax.experimental import pallas as pl
from jax.experimental.pallas import tpu as pltpu
from jax.experimental.pallas import tpu_sc as plsc

HID = 256
LN_EPS = 1e-5


def _mlp_tail(h, w2_ref, b2_ref, gm_ref, bt_ref):
    h = h * jax.nn.sigmoid(h)
    h = jnp.dot(h, w2_ref[...], preferred_element_type=jnp.float32) + b2_ref[...]
    if gm_ref is not None:
        mu = jnp.mean(h, axis=-1, keepdims=True)
        var = jnp.mean((h - mu) ** 2, axis=-1, keepdims=True)
        h = (h - mu) * lax.rsqrt(var + LN_EPS) * gm_ref[...] + bt_ref[...]
    return h


def _embed_body(x_ref, w1_ref, b1_ref, w2_ref, b2_ref, gm_ref, bt_ref, o_ref):
    h = jnp.dot(x_ref[...], w1_ref[...], preferred_element_type=jnp.float32)
    h = h + b1_ref[...]
    o_ref[...] = _mlp_tail(h, w2_ref, b2_ref, gm_ref, bt_ref)


def _pre_body(nf_ref, ws_ref, wr_ref, ns_ref, nr_ref):
    nf = nf_ref[...]
    ns_ref[...] = jnp.dot(nf, ws_ref[...], preferred_element_type=jnp.float32)
    nr_ref[...] = jnp.dot(nf, wr_ref[...], preferred_element_type=jnp.float32)


def _edge_body(ef_ref, g_ref, w1_ref, b1_ref, w2_ref, b2_ref, gm_ref, bt_ref,
               ne_ref, efo_ref):
    ef = ef_ref[...]
    h = jnp.dot(ef, w1_ref[...], preferred_element_type=jnp.float32)
    h = h + g_ref[...] + b1_ref[...]
    h = _mlp_tail(h, w2_ref, b2_ref, gm_ref, bt_ref)
    ne_ref[...] = h
    efo_ref[...] = ef + h


def _node_body(nf_ref, agg_ref, w1a_ref, w1b_ref, b1_ref, w2_ref, b2_ref,
               gm_ref, bt_ref, ws_ref, wr_ref, nfo_ref, ns_ref, nr_ref):
    nf = nf_ref[...]
    h = jnp.dot(nf, w1a_ref[...], preferred_element_type=jnp.float32)
    h = h + jnp.dot(agg_ref[...], w1b_ref[...], preferred_element_type=jnp.float32)
    h = h + b1_ref[...]
    h = _mlp_tail(h, w2_ref, b2_ref, gm_ref, bt_ref)
    nfo = nf + h
    nfo_ref[...] = nfo
    ns_ref[...] = jnp.dot(nfo, ws_ref[...], preferred_element_type=jnp.float32)
    nr_ref[...] = jnp.dot(nfo, wr_ref[...], preferred_element_type=jnp.float32)


def _out_body(nf_ref, w1_ref, b1_ref, w2_ref, b2_ref, o_ref):
    h = jnp.dot(nf_ref[...], w1_ref[...], preferred_element_type=jnp.float32)
    h = h + b1_ref[...]
    o_ref[...] = _mlp_tail(h, w2_ref, b2_ref, None, None)


def _row_spec(blk, d):
    return pl.BlockSpec((blk, d), lambda i: (i, 0))


def _full_spec(shape):
    nd = len(shape)
    return pl.BlockSpec(shape, lambda i: (0,) * nd)


def _pick_block(n, want):
    if n % want == 0:
        return want
    b = min(n, want)
    while n % b != 0:
        b -= 1
    return b


def _embed_mlp(x, p):
    e, d_in = x.shape
    blk = _pick_block(e, 2000)
    return pl.pallas_call(
        _embed_body,
        grid=(e // blk,),
        in_specs=[
            _row_spec(blk, d_in),
            _full_spec((d_in, HID)), _full_spec((1, HID)),
            _full_spec((HID, HID)), _full_spec((1, HID)),
            _full_spec((1, HID)), _full_spec((1, HID)),
        ],
        out_specs=_row_spec(blk, HID),
        out_shape=jax.ShapeDtypeStruct((e, HID), jnp.float32),
    )(x, p['W1'], p['b1'].reshape(1, -1), p['W2'], p['b2'].reshape(1, -1),
      p['g'].reshape(1, -1), p['bt'].reshape(1, -1))


def _pre_transform(nf, ws, wr):
    n = nf.shape[0]
    blk = _pick_block(n, 2000)
    return pl.pallas_call(
        _pre_body,
        grid=(n // blk,),
        in_specs=[_row_spec(blk, HID), _full_spec((HID, HID)),
                  _full_spec((HID, HID))],
        out_specs=[_row_spec(blk, HID), _row_spec(blk, HID)],
        out_shape=[jax.ShapeDtypeStruct((n, HID), jnp.float32),
                   jax.ShapeDtypeStruct((n, HID), jnp.float32)],
    )(nf, ws, wr)


def _edge_step(ef, g, w1e, p):
    e = ef.shape[0]
    blk = _pick_block(e, 2000)
    return pl.pallas_call(
        _edge_body,
        grid=(e // blk,),
        in_specs=[
            _row_spec(blk, HID), _row_spec(blk, HID),
            _full_spec((HID, HID)), _full_spec((1, HID)),
            _full_spec((HID, HID)), _full_spec((1, HID)),
            _full_spec((1, HID)), _full_spec((1, HID)),
        ],
        out_specs=[_row_spec(blk, HID), _row_spec(blk, HID)],
        out_shape=[jax.ShapeDtypeStruct((e, HID), jnp.float32),
                   jax.ShapeDtypeStruct((e, HID), jnp.float32)],
    )(ef, g, w1e, p['b1'].reshape(1, -1), p['W2'], p['b2'].reshape(1, -1),
      p['g'].reshape(1, -1), p['bt'].reshape(1, -1))


def _node_step(nf, agg, p, ws_next, wr_next):
    n = nf.shape[0]
    blk = _pick_block(n, 2000)
    w1a = p['W1'][:HID]
    w1b = p['W1'][HID:]
    return pl.pallas_call(
        _node_body,
        grid=(n // blk,),
        in_specs=[
            _row_spec(blk, HID), _row_spec(blk, HID),
            _full_spec((HID, HID)), _full_spec((HID, HID)),
            _full_spec((1, HID)),
            _full_spec((HID, HID)), _full_spec((1, HID)),
            _full_spec((1, HID)), _full_spec((1, HID)),
            _full_spec((HID, HID)), _full_spec((HID, HID)),
        ],
        out_specs=[_row_spec(blk, HID), _row_spec(blk, HID),
                   _row_spec(blk, HID)],
        out_shape=[jax.ShapeDtypeStruct((n, HID), jnp.float32),
                   jax.ShapeDtypeStruct((n, HID), jnp.float32),
                   jax.ShapeDtypeStruct((n, HID), jnp.float32)],
    )(nf, agg, w1a, w1b, p['b1'].reshape(1, -1), p['W2'],
      p['b2'].reshape(1, -1), p['g'].reshape(1, -1), p['bt'].reshape(1, -1),
      ws_next, wr_next)


def _out_mlp(nf, p):
    n = nf.shape[0]
    blk = _pick_block(n, 2000)
    return pl.pallas_call(
        _out_body,
        grid=(n // blk,),
        in_specs=[
            _row_spec(blk, HID),
            _full_spec((HID, HID)), _full_spec((1, HID)),
            _full_spec((HID, HID)), _full_spec((1, HID)),
        ],
        out_specs=_row_spec(blk, HID),
        out_shape=jax.ShapeDtypeStruct((n, HID), jnp.float32),
    )(nf, p['W1'], p['b1'].reshape(1, -1), p['W2'], p['b2'].reshape(1, -1))


_NC = 2
_NS = 16
_NW = _NC * _NS
_LANES = 16
_CHUNK = 80


def _vmem_add(dst_ref, src_ref, rows):
    def body(r, _):
        for k in range(HID // _LANES):
            sl = pl.ds(k * _LANES, _LANES)
            dst_ref[r, sl] = dst_ref[r, sl] + src_ref[r, sl]
        return 0
    lax.fori_loop(0, rows, body, 0, unroll=False)


def _gather_add(ns, nr, send3, recv3, e, per_w, n_ch, tail):
    mesh = plsc.VectorSubcoreMesh(core_axis_name="c", subcore_axis_name="s")

    @functools.partial(
        pl.kernel, mesh=mesh,
        out_type=jax.ShapeDtypeStruct((e, HID), jnp.float32),
        scratch_types=[
            pltpu.VMEM((n_ch, _CHUNK), jnp.int32),
            pltpu.VMEM((n_ch, _CHUNK), jnp.int32),
            pltpu.VMEM((_CHUNK, HID), jnp.float32),
            pltpu.VMEM((_CHUNK, HID), jnp.float32),
            pltpu.VMEM((_CHUNK, HID), jnp.float32),
            pltpu.VMEM((_CHUNK, HID), jnp.float32),
            pltpu.SemaphoreType.DMA,
            pltpu.SemaphoreType.DMA,
            pltpu.SemaphoreType.DMA,
            pltpu.SemaphoreType.DMA,
        ],
    )
    def gather_kernel(ns_hbm, nr_hbm, send_hbm, recv_hbm, g_hbm,
                      iv_s, iv_r, ra0, rb0, ra1, rb1, s0, s1, s2, s3):
        wid = lax.axis_index("s") * _NC + lax.axis_index("c")
        base = wid * per_w
        bufs = ((ra0, rb0, s0, s1), (ra1, rb1, s2, s3))

        pltpu.sync_copy(send_hbm.at[wid], iv_s)
        pltpu.sync_copy(recv_hbm.at[wid], iv_r)

        def issue(j, b):
            ra, rb, sa, sb = bufs[b]
            pltpu.async_copy(ns_hbm.at[iv_s.at[j]], ra, sa)
            pltpu.async_copy(nr_hbm.at[iv_r.at[j]], rb, sb)

        def consume(j, b, rows):
            ra, rb, sa, sb = bufs[b]
            pltpu.make_async_copy(ns_hbm.at[iv_s.at[j]], ra, sa).wait()
            pltpu.make_async_copy(nr_hbm.at[iv_r.at[j]], rb, sb).wait()
            _vmem_add(ra, rb, rows)
            if rows == _CHUNK:
                pltpu.sync_copy(ra, g_hbm.at[pl.ds(base + j * _CHUNK, _CHUNK)])
            else:
                pltpu.sync_copy(ra.at[pl.ds(0, rows)],
                                g_hbm.at[pl.ds(base + j * _CHUNK, rows)])

        issue(0, 0)
        issue(1, 1)

        def pair(jj, _):
            j0 = 2 * jj
            consume(j0, 0, _CHUNK)
            @pl.when(j0 + 2 < n_ch)
            def _():
                issue(j0 + 2, 0)
            consume(j0 + 1, 1, _CHUNK)
            @pl.when(j0 + 3 < n_ch)
            def _():
                issue(j0 + 3, 1)
            return 0

        lax.fori_loop(0, (n_ch - 1) // 2, pair, 0, unroll=False)
        consume(n_ch - 1, (n_ch - 1) % 2, tail)

    return gather_kernel(ns, nr, send3, recv3)


_N_PAD = 10240


def _seg_cumsum(ne, order3, e, per_w, n_ch, tail):
    ngr = HID // _LANES
    mesh = plsc.VectorSubcoreMesh(core_axis_name="c", subcore_axis_name="s")

    @functools.partial(
        pl.kernel, mesh=mesh,
        out_type=[jax.ShapeDtypeStruct((e + 8, HID), jnp.float32),
                  jax.ShapeDtypeStruct((_NW, HID), jnp.float32)],
        scratch_types=[
            pltpu.VMEM((n_ch, _CHUNK), jnp.int32),
            pltpu.VMEM((_CHUNK, HID), jnp.float32),
            pltpu.VMEM((_CHUNK, HID), jnp.float32),
            pltpu.VMEM((_CHUNK, HID), jnp.float32),
            pltpu.VMEM((1, HID), jnp.float32),
            pltpu.SemaphoreType.DMA,
            pltpu.SemaphoreType.DMA,
        ],
    )
    def cumsum_kernel(ne_hbm, ord_hbm, cs_hbm, tot_hbm,
                      iv, rin0, rin1, rout, carry_v, s0, s1):
        wid = lax.axis_index("s") * _NC + lax.axis_index("c")
        base = wid * per_w
        bufs = ((rin0, s0), (rin1, s1))

        pltpu.sync_copy(ord_hbm.at[wid], iv)

        zero = jnp.zeros((_LANES,), jnp.float32)
        for k in range(ngr):
            carry_v[0, pl.ds(k * _LANES, _LANES)] = zero

        def issue(j, b):
            rin, sa = bufs[b]
            pltpu.async_copy(ne_hbm.at[iv.at[j]], rin, sa)

        def consume(j, b, rows):
            rin, sa = bufs[b]
            pltpu.make_async_copy(ne_hbm.at[iv.at[j]], rin, sa).wait()
            for k0 in range(0, ngr, 4):
                sls = [pl.ds(k * _LANES, _LANES) for k in range(k0, k0 + 4)]

                def rowbody(r, cvs):
                    new = []
                    for sl, cv in zip(sls, cvs):
                        rout[r, sl] = cv
                        new.append(cv + rin[r, sl])
                    return tuple(new)

                cvs = lax.fori_loop(
                    0, rows, rowbody,
                    tuple(carry_v[0, sl] for sl in sls), unroll=4)
                for sl, cv in zip(sls, cvs):
                    carry_v[0, sl] = cv
            if rows == _CHUNK:
                pltpu.sync_copy(rout, cs_hbm.at[pl.ds(base + j * _CHUNK,
                                                      _CHUNK)])
            else:
                pltpu.sync_copy(rout.at[pl.ds(0, rows)],
                                cs_hbm.at[pl.ds(base + j * _CHUNK, rows)])

        issue(0, 0)
        issue(1, 1)

        def pair(jj, _):
            j0 = 2 * jj
            consume(j0, 0, _CHUNK)
            @pl.when(j0 + 2 < n_ch)
            def _():
                issue(j0 + 2, 0)
            consume(j0 + 1, 1, _CHUNK)
            @pl.when(j0 + 3 < n_ch)
            def _():
                issue(j0 + 3, 1)
            return 0

        lax.fori_loop(0, (n_ch - 1) // 2, pair, 0, unroll=False)
        consume(n_ch - 1, (n_ch - 1) % 2, tail)

        pltpu.sync_copy(carry_v, tot_hbm.at[pl.ds(wid, 1)])
        @pl.when(wid == 0)
        def _():
            def zrow(r, _):
                for k in range(ngr):
                    rout[r, pl.ds(k * _LANES, _LANES)] = zero
                return 0
            lax.fori_loop(0, 8, zrow, 0, unroll=False)
            pltpu.sync_copy(rout.at[pl.ds(0, 8)], cs_hbm.at[pl.ds(e, 8)])

    return cumsum_kernel(ne, order3)


def _offsets_body(tot_ref, off_ref):
    acc = jnp.zeros((1, HID), jnp.float32)
    for t in range(_NW):
        off_ref[pl.ds(t, 1), :] = acc
        acc = acc + tot_ref[pl.ds(t, 1), :]
    for t in range(_NW, 40):
        off_ref[pl.ds(t, 1), :] = acc


def _seg_offsets(tot):
    return pl.pallas_call(
        _offsets_body,
        grid=(1,),
        in_specs=[_full_spec((_NW, HID))],
        out_specs=_full_spec((40, HID)),
        out_shape=jax.ShapeDtypeStruct((40, HID), jnp.float32),
    )(tot)


def _seg_boundary(cs, off, rp_start, rp_end, per_w):
    nodes_per_t = _N_PAD // _NW
    n_sub = nodes_per_t // _CHUNK
    ngr = HID // _LANES

    mesh = plsc.VectorSubcoreMesh(core_axis_name="c", subcore_axis_name="s")

    @functools.partial(
        pl.kernel, mesh=mesh,
        out_type=jax.ShapeDtypeStruct((_N_PAD, HID), jnp.float32),
        scratch_types=[
            pltpu.VMEM((_CHUNK,), jnp.int32),
            pltpu.VMEM((_CHUNK,), jnp.int32),
            pltpu.VMEM((_CHUNK,), jnp.int32),
            pltpu.VMEM((_CHUNK,), jnp.int32),
            pltpu.VMEM((_CHUNK, HID), jnp.float32),
            pltpu.VMEM((_CHUNK, HID), jnp.float32),
            pltpu.VMEM((_CHUNK, HID), jnp.float32),
            pltpu.VMEM((_CHUNK, HID), jnp.float32),
            pltpu.SemaphoreType.DMA,
            pltpu.SemaphoreType.DMA,
            pltpu.SemaphoreType.DMA,
            pltpu.SemaphoreType.DMA,
        ],
    )
    def boundary_kernel(cs_hbm, off_hbm, rs_hbm, re_hbm, agg_hbm,
                        ia, ie, ta, te, ga, gb, oa, ob, s0, s1, s2, s3):
        wid = lax.axis_index("s") * _NC + lax.axis_index("c")
        pw = jnp.full((_LANES,), per_w, jnp.int32)

        def sub(q, _):
            nbase = wid * nodes_per_t + q * _CHUNK
            pltpu.sync_copy(rs_hbm.at[pl.ds(nbase, _CHUNK)], ia)
            pltpu.sync_copy(re_hbm.at[pl.ds(nbase, _CHUNK)], ie)
            for g in range(_CHUNK // _LANES):
                sl = pl.ds(g * _LANES, _LANES)
                ta[sl] = lax.div(ia[sl], pw)
                te[sl] = lax.div(ie[sl], pw)
            d0 = pltpu.async_copy(cs_hbm.at[ia], ga, s0)
            d1 = pltpu.async_copy(cs_hbm.at[ie], gb, s1)
            d2 = pltpu.async_copy(off_hbm.at[ta], oa, s2)
            d3 = pltpu.async_copy(off_hbm.at[te], ob, s3)
            d0.wait()
            d1.wait()
            d2.wait()
            d3.wait()

            def row(r, _):
                for k in range(ngr):
                    slk = pl.ds(k * _LANES, _LANES)
                    ga[r, slk] = (gb[r, slk] + ob[r, slk]) - (
                        ga[r, slk] + oa[r, slk])
                return 0
            lax.fori_loop(0, _CHUNK, row, 0, unroll=False)
            pltpu.sync_copy(ga, agg_hbm.at[pl.ds(nbase, _CHUNK)])
            return 0

        lax.fori_loop(0, n_sub, sub, 0, unroll=False)

    return boundary_kernel(cs, off, rp_start, rp_end)


def _segment_sum(ne, order3, rp_start, rp_end, e, per_w, n_ch, tail):
    cs, tot = _seg_cumsum(ne, order3, e, per_w, n_ch, tail)
    off = _seg_offsets(tot)
    return _seg_boundary(cs, off, rp_start, rp_end, per_w)


def kernel(edge_idx, edge_features, node_features, params):
    send = edge_idx[0]
    recv = edge_idx[1]
    n = node_features.shape[0]

    e = send.shape[0]
    per_w = e // _NW
    n_ch = -(-per_w // _CHUNK)
    tail = per_w - (n_ch - 1) * _CHUNK
    pad = n_ch * _CHUNK - per_w

    def _blocks(idx):
        return jnp.pad(idx.reshape(_NW, per_w),
                       ((0, 0), (0, pad))).reshape(_NW, n_ch, _CHUNK)

    keys = (recv.astype(jnp.uint32) << 18) | jnp.arange(e, dtype=jnp.uint32)
    skeys = jnp.sort(keys)
    order = (skeys & jnp.uint32((1 << 18) - 1)).astype(jnp.int32)
    recv_sorted = (skeys >> 18).astype(jnp.int32)
    rowptr = jnp.searchsorted(
        recv_sorted, jnp.arange(_N_PAD + 1, dtype=jnp.int32)).astype(jnp.int32)
    rp_start = rowptr[:_N_PAD]
    rp_end = rowptr[1:]
    send3 = _blocks(send)
    recv3 = _blocks(recv)
    order3 = _blocks(order)

    ef = _embed_mlp(edge_features, params['embed_edge'])
    nf = node_features
    w1 = params['step0']['edge']['W1']
    ns, nr = _pre_transform(nf, w1[HID:2 * HID], w1[2 * HID:])
    for i in range(4):
        p = params['step%d' % i]
        w1 = p['edge']['W1']
        g = _gather_add(ns, nr, send3, recv3, e, per_w, n_ch, tail)
        ne, ef = _edge_step(ef, g, w1[:HID], p['edge'])
        agg = _segment_sum(ne, order3, rp_start, rp_end,
                           e, per_w, n_ch, tail)[:n]
        w1n = params['step%d' % min(i + 1, 3)]['edge']['W1']
        nf, ns, nr = _node_step(nf, agg, p['node'],
                                w1n[HID:2 * HID], w1n[2 * HID:])
    nf = _out_mlp(nf, params['node_out'])
    return (ef, nf)
